# baseline (speedup 1.0000x reference)
import functools

import jax
import jax.numpy as jnp
from jax import lax
from jax.experimental import pallas as pl
from jax.experimental.pallas import tpu as pltpu
from jax.experimental.pallas import tpu_sc as plsc

BATCH = 64
SEQ = 512
EMBD = 512
NW = 32
PW = SEQ // NW
LANES = 16
VECS = EMBD // LANES
BBLK = BATCH // LANES


def _emb_body(x_hbm, tok_hbm, pos_hbm, out_hbm,
              pos_v, tokid_v, oidx2_v, oidxf_v,
              rows0, rows1, rows2, psem,
              gsem0, gsem1, gsem2, ssem0, ssem1, ssem2):
    wid = lax.axis_index("s") * 2 + lax.axis_index("c")
    p0 = wid * PW
    ph = pltpu.async_copy(pos_hbm.at[pl.ds(p0, PW)], pos_v, psem)

    bvec = lax.iota(jnp.int32, LANES) * SEQ
    for j in range(PW):
        for kk in range(BBLK):
            val = bvec + (kk * LANES * SEQ + p0 + j)
            sl = pl.ds(kk * LANES, LANES)
            oidx2_v[j, sl] = val
            oidxf_v[pl.ds(j * BATCH + kk * LANES, LANES)] = val
    pltpu.sync_copy(x_hbm.at[oidxf_v], tokid_v)

    rows = (rows0, rows1, rows2)
    gsem = (gsem0, gsem1, gsem2)
    ssem = (ssem0, ssem1, ssem2)
    NB = 3

    def start_gather(j, buf):
        return pltpu.async_copy(
            tok_hbm.at[tokid_v.at[pl.ds(j * BATCH, BATCH)]], rows[buf],
            gsem[buf])

    def add_pos(j, buf):
        r = rows[buf]
        pv = [pos_v[j, pl.ds(k * LANES, LANES)] for k in range(VECS)]

        def body(rr, carry):
            for u in range(2):
                row = rr * 2 + u
                for k in range(VECS):
                    sl = pl.ds(k * LANES, LANES)
                    r[row, sl] = r[row, sl] + pv[k]
            return carry

        lax.fori_loop(0, BATCH // 2, body, 0)

    def start_scatter(j, buf):
        return pltpu.async_copy(rows[buf], out_hbm.at[oidx2_v.at[j]],
                                ssem[buf])

    g = [None] * PW
    s = [None] * PW
    g[0] = start_gather(0, 0)
    ph.wait()
    for j in range(PW):
        buf = j % NB
        if j + 1 < PW:
            if j >= 2:
                s[j - 2].wait()
            g[j + 1] = start_gather(j + 1, (j + 1) % NB)
        g[j].wait()
        add_pos(j, buf)
        s[j] = start_scatter(j, buf)
    s[PW - 3].wait()
    s[PW - 2].wait()
    s[PW - 1].wait()


def kernel(x, token_table, position_table):
    xf = x.reshape(-1).astype(jnp.int32)
    mesh = plsc.VectorSubcoreMesh(core_axis_name="c", subcore_axis_name="s")
    f = functools.partial(
        pl.kernel,
        mesh=mesh,
        out_type=jax.ShapeDtypeStruct((BATCH * SEQ, EMBD), jnp.float32),
        scratch_types=[
            pltpu.VMEM((PW, EMBD), jnp.float32),
            pltpu.VMEM((PW * BATCH,), jnp.int32),
            pltpu.VMEM((PW, BATCH), jnp.int32),
            pltpu.VMEM((PW * BATCH,), jnp.int32),
            pltpu.VMEM((BATCH, EMBD), jnp.float32),
            pltpu.VMEM((BATCH, EMBD), jnp.float32),
            pltpu.VMEM((BATCH, EMBD), jnp.float32),
            pltpu.SemaphoreType.DMA,
            pltpu.SemaphoreType.DMA,
            pltpu.SemaphoreType.DMA,
            pltpu.SemaphoreType.DMA,
            pltpu.SemaphoreType.DMA,
            pltpu.SemaphoreType.DMA,
            pltpu.SemaphoreType.DMA,
        ],
    )(_emb_body)
    out = f(xf, token_table, position_table)
    return out.reshape(BATCH, SEQ, EMBD)


# speedup vs baseline: 1.2072x; 1.2072x over previous
import functools

import jax
import jax.numpy as jnp
from jax import lax
from jax.experimental import pallas as pl
from jax.experimental.pallas import tpu as pltpu
from jax.experimental.pallas import tpu_sc as plsc

BATCH = 64
SEQ = 512
EMBD = 512
NW = 32
PW = SEQ // NW
LANES = 16
VECS = EMBD // LANES
BBLK = BATCH // LANES


def _emb_body(x_hbm, tok_hbm, pos_hbm, out_hbm,
              pos_v, tokid_v, oidx2_v, oidxf_v,
              rows0, rows1, rows2, psem,
              gsem0, gsem1, gsem2, ssem0, ssem1, ssem2):
    wid = lax.axis_index("s") * 2 + lax.axis_index("c")
    p0 = wid * PW
    ph = pltpu.async_copy(pos_hbm.at[pl.ds(p0, PW)], pos_v, psem)

    bvec = lax.iota(jnp.int32, LANES) * SEQ
    for j in range(PW):
        for kk in range(BBLK):
            val = bvec + (kk * LANES * SEQ + p0 + j)
            sl = pl.ds(kk * LANES, LANES)
            oidx2_v[j, sl] = val
            oidxf_v[pl.ds(j * BATCH + kk * LANES, LANES)] = val
    pltpu.sync_copy(x_hbm.at[oidxf_v], tokid_v)

    rows = (rows0, rows1, rows2)
    gsem = (gsem0, gsem1, gsem2)
    ssem = (ssem0, ssem1, ssem2)
    NB = 3

    def start_gather(j, buf):
        return pltpu.async_copy(
            tok_hbm.at[tokid_v.at[pl.ds(j * BATCH, BATCH)]], rows[buf],
            gsem[buf])

    def add_pos(j, buf):
        r = rows[buf]
        pv = [pos_v[j, pl.ds(k * LANES, LANES)] for k in range(VECS)]

        def body(row, carry):
            for k in range(VECS):
                sl = pl.ds(k * LANES, LANES)
                r[row, sl] = r[row, sl] + pv[k]
            return carry

        lax.fori_loop(0, BATCH, body, 0)

    def start_scatter(j, buf):
        return pltpu.async_copy(rows[buf], out_hbm.at[oidx2_v.at[j]],
                                ssem[buf])

    g = [None] * PW
    s = [None] * PW
    g[0] = start_gather(0, 0)
    ph.wait()
    for j in range(PW):
        buf = j % NB
        if j + 1 < PW:
            if j >= 2:
                s[j - 2].wait()
            g[j + 1] = start_gather(j + 1, (j + 1) % NB)
        g[j].wait()
        add_pos(j, buf)
        s[j] = start_scatter(j, buf)
    s[PW - 3].wait()
    s[PW - 2].wait()
    s[PW - 1].wait()


def kernel(x, token_table, position_table):
    xf = x.reshape(-1).astype(jnp.int32)
    mesh = plsc.VectorSubcoreMesh(core_axis_name="c", subcore_axis_name="s")
    f = functools.partial(
        pl.kernel,
        mesh=mesh,
        out_type=jax.ShapeDtypeStruct((BATCH * SEQ, EMBD), jnp.float32),
        scratch_types=[
            pltpu.VMEM((PW, EMBD), jnp.float32),
            pltpu.VMEM((PW * BATCH,), jnp.int32),
            pltpu.VMEM((PW, BATCH), jnp.int32),
            pltpu.VMEM((PW * BATCH,), jnp.int32),
            pltpu.VMEM((BATCH, EMBD), jnp.float32),
            pltpu.VMEM((BATCH, EMBD), jnp.float32),
            pltpu.VMEM((BATCH, EMBD), jnp.float32),
            pltpu.SemaphoreType.DMA,
            pltpu.SemaphoreType.DMA,
            pltpu.SemaphoreType.DMA,
            pltpu.SemaphoreType.DMA,
            pltpu.SemaphoreType.DMA,
            pltpu.SemaphoreType.DMA,
            pltpu.SemaphoreType.DMA,
        ],
    )(_emb_body)
    out = f(xf, token_table, position_table)
    return out.reshape(BATCH, SEQ, EMBD)
